# baseline (speedup 1.0000x reference)
import functools

import jax
import jax.numpy as jnp
from jax import lax
from jax.experimental import pallas as pl
from jax.experimental.pallas import tpu as pltpu
from jax.experimental.pallas import tpu_sc as plsc

N_ATOMS = 10000
H = 128
NBR = 32
N_MOLS = 200
MOL_SIZE = 50
FFN_HID = 512
MOL_HID = 256
NUM_TASKS = 12

CHUNK_ATOMS = 4
ROWS = CHUNK_ATOMS * NBR
NPAD = 10560
NCHT = NPAD // CHUNK_ATOMS
NCHPAD = NCHT + 128
B_BASE_CHUNK = 2000
NBUF = 2
A_APW = (416, 96)
B_APW = (128, 32)


def _make_sc_gather_sum(apw0, apw1, call_base_chunk):
    nch0, nch1 = apw0 // CHUNK_ATOMS, apw1 // CHUNK_ATOMS
    nout = 16 * (apw0 + apw1)
    mesh = plsc.VectorSubcoreMesh(core_axis_name="c", subcore_axis_name="s")

    @functools.partial(
        pl.kernel,
        mesh=mesh,
        out_type=jax.ShapeDtypeStruct((nout, H), jnp.float32),
        scratch_types=[
            pltpu.VMEM((nch0, ROWS), jnp.int32),
            pltpu.VMEM_SHARED((16 * apw0, H), jnp.float32),
        ]
        + [pltpu.VMEM((ROWS, H), jnp.float32)] * NBUF
        + [pltpu.VMEM((ROWS,), jnp.int32)] * NBUF
        + [pltpu.SemaphoreType.DMA] * (2 * NBUF),
    )
    def k(idx_hbm, table_hbm, out_hbm, idx_v, acc_sh, *rest):
        rows = rest[:NBUF]
        own = rest[NBUF:2 * NBUF]
        sem_g = rest[2 * NBUF:3 * NBUF]
        sem_s = rest[3 * NBUF:]
        c = lax.axis_index("c")
        s = lax.axis_index("s")
        apw = jnp.where(c == 0, apw0, apw1)
        nch = jnp.where(c == 0, nch0, nch1)
        base_local = s * apw
        base_chunk = call_base_chunk + jnp.where(
            c == 0, s * nch0, 16 * nch0 + s * nch1)
        pltpu.sync_copy(idx_hbm.at[pl.ds(base_chunk, nch0)], idx_v)

        zero = jnp.zeros((16,), jnp.float32)

        def zbuf(r, carry):
            for g in range(H // 16):
                rows[0][r, pl.ds(g * 16, 16)] = zero
            return carry

        lax.fori_loop(0, ROWS, zbuf, 0)

        @pl.when(c == 0)
        def _():
            for t in range(apw0 // ROWS):
                pltpu.sync_copy(
                    rows[0], acc_sh.at[pl.ds(s * apw0 + t * ROWS, ROWS)])
            rem = apw0 % ROWS
            if rem:
                pltpu.sync_copy(
                    rows[0].at[pl.ds(0, rem)],
                    acc_sh.at[pl.ds(s * apw0 + (apw0 // ROWS) * ROWS, rem)])

        @pl.when(c != 0)
        def _():
            for t in range(apw1 // ROWS):
                pltpu.sync_copy(
                    rows[0], acc_sh.at[pl.ds(s * apw1 + t * ROWS, ROWS)])
            rem = apw1 % ROWS
            if rem:
                pltpu.sync_copy(
                    rows[0].at[pl.ds(0, rem)],
                    acc_sh.at[pl.ds(s * apw1 + (apw1 // ROWS) * ROWS, rem)])

        def gather(ci, b):
            pltpu.async_copy(table_hbm.at[idx_v.at[ci]], rows[b], sem_g[b])

        def wait_gather(ci, b):
            pltpu.make_async_copy(table_hbm.at[idx_v.at[ci]], rows[b],
                                  sem_g[b]).wait()

        def scat(b):
            pltpu.async_copy(rows[b], acc_sh.at[own[b]], sem_s[b], add=True)

        def wait_scat(b):
            pltpu.make_async_copy(rows[b], acc_sh.at[own[b]], sem_s[b]).wait()

        for b in range(NBUF):
            gather(b, b)

        def pair(p, carry):
            for b in range(NBUF):
                ci = p * NBUF + b
                wait_gather(ci, b)
                for g in range(ROWS // 16):
                    own[b][pl.ds(g * 16, 16)] = jnp.full(
                        (16,), ci * CHUNK_ATOMS + g // 2, jnp.int32
                    ) + base_local
                scat(b)
            for b in range(NBUF):
                @pl.when(p * NBUF + b + NBUF < nch)
                def _(b=b):
                    ci = p * NBUF + b
                    wait_scat(b)
                    gather(ci + NBUF, b)
            return carry

        lax.fori_loop(0, nch // NBUF, pair, 0)
        for b in range(NBUF):
            wait_scat(b)

        @pl.when(c == 0)
        def _():
            pltpu.sync_copy(acc_sh.at[pl.ds(s * apw0, apw0)],
                            out_hbm.at[pl.ds(s * apw0, apw0)])

        @pl.when(c != 0)
        def _():
            pltpu.sync_copy(acc_sh.at[pl.ds(s * apw1, apw1)],
                            out_hbm.at[pl.ds(16 * apw0 + s * apw1, apw1)])

    return k


ROWS_BLK = 400
NBLK = N_ATOMS // ROWS_BLK
MOLS_BLK = ROWS_BLK // MOL_SIZE


AB_BLK = 20


def _tc_body(orig_ref, aggr_a_ref, aggr_b_ref, w1a_ref, w1b_ref, b1_ref,
             w2_ref, b2_ref, g_ref, b_ref, wm1_ref, bm1_ref, wm2_ref, bm2_ref,
             out_ref, macc):
    i = pl.program_id(0)
    aggr = jnp.where(i < AB_BLK, aggr_a_ref[...], aggr_b_ref[...])
    x = jnp.dot(orig_ref[...], w1a_ref[...], preferred_element_type=jnp.float32)
    x = x + jnp.dot(aggr, w1b_ref[...], preferred_element_type=jnp.float32)
    h = jnp.maximum(x + b1_ref[...], 0.0)
    y = jnp.dot(h, w2_ref[...], preferred_element_type=jnp.float32) + b2_ref[...]
    mu = jnp.mean(y, axis=-1, keepdims=True)
    var = jnp.mean((y - mu) * (y - mu), axis=-1, keepdims=True)
    yn = (y - mu) * lax.rsqrt(var + 1e-5) * g_ref[...] + b_ref[...]
    pooled = jnp.sum(yn.reshape(MOLS_BLK, MOL_SIZE, H), axis=1) * (1.0 / MOL_SIZE)
    macc[pl.ds(i * MOLS_BLK, MOLS_BLK), :] = pooled

    @pl.when(i == NBLK - 1)
    def _():
        m = macc[...]
        hm = jnp.maximum(
            jnp.dot(m, wm1_ref[...], preferred_element_type=jnp.float32)
            + bm1_ref[...], 0.0)
        out = jnp.dot(hm, wm2_ref[...], preferred_element_type=jnp.float32)
        out_ref[...] = (out + bm2_ref[...]) * 0.5


def _tc_ffn(orig, aggr_a, aggr_b, w1a, w1b, b1, w2, b2, g, b, wm1, bm1, wm2,
            bm2):
    full = lambda shape: pl.BlockSpec(shape, lambda i: (0, 0))
    return pl.pallas_call(
        _tc_body,
        grid=(NBLK,),
        in_specs=[
            pl.BlockSpec((ROWS_BLK, H), lambda i: (i, 0)),
            pl.BlockSpec((ROWS_BLK, H), lambda i: (jnp.minimum(i, AB_BLK - 1), 0)),
            pl.BlockSpec((ROWS_BLK, H),
                         lambda i: (jnp.maximum(i - AB_BLK, 0), 0)),
            full((H, FFN_HID)),
            full((H, FFN_HID)),
            full((1, FFN_HID)),
            full((FFN_HID, H)),
            full((1, H)),
            full((1, H)),
            full((1, H)),
            full((H, MOL_HID)),
            full((1, MOL_HID)),
            full((MOL_HID, H)),
            full((1, H)),
        ],
        out_specs=pl.BlockSpec((N_MOLS, H), lambda i: (0, 0)),
        out_shape=jax.ShapeDtypeStruct((N_MOLS, H), jnp.float32),
        scratch_shapes=[pltpu.VMEM((N_MOLS, H), jnp.float32)],
    )(orig, aggr_a, aggr_b, w1a, w1b, b1, w2, b2, g, b, wm1, bm1, wm2, bm2)


def kernel(atom_output, bond_output, original_f_atoms, original_f_bonds,
           a2a, a2b, b2a, b2revb, a_scope,
           W_aa1, b_aa1, W_aa2, b_aa2, ln_aa_g, ln_aa_b,
           W_ab1, b_ab1, W_ab2, b_ab2, ln_ab_g, ln_ab_b,
           W_m1, b_m1, W_m2, b_m2):
    idx = jnp.zeros((NCHPAD, ROWS), jnp.int32).at[:NCHT].set(
        jnp.pad(a2a.astype(jnp.int32),
                ((0, NPAD - N_ATOMS), (0, 0))).reshape(NCHT, ROWS))
    aggr_a = _make_sc_gather_sum(*A_APW, 0)(idx, atom_output)
    aggr_b = _make_sc_gather_sum(*B_APW, B_BASE_CHUNK)(idx, atom_output)

    w1a = W_aa1[:H]
    w1b = W_aa1[H:]
    wm2 = jnp.zeros((MOL_HID, H), jnp.float32).at[:, :NUM_TASKS].set(W_m2)
    bm2 = jnp.zeros((1, H), jnp.float32).at[0, :NUM_TASKS].set(b_m2)
    out = _tc_ffn(original_f_atoms, aggr_a, aggr_b, w1a, w1b,
                  b_aa1.reshape(1, -1), W_aa2, b_aa2.reshape(1, -1),
                  ln_aa_g.reshape(1, -1), ln_aa_b.reshape(1, -1),
                  W_m1, b_m1.reshape(1, -1), wm2, bm2)
    return out[:, :NUM_TASKS]


# speedup vs baseline: 1.0014x; 1.0014x over previous
import functools

import jax
import jax.numpy as jnp
from jax import lax
from jax.experimental import pallas as pl
from jax.experimental.pallas import tpu as pltpu
from jax.experimental.pallas import tpu_sc as plsc

N_ATOMS = 10000
H = 128
NBR = 32
N_MOLS = 200
MOL_SIZE = 50
FFN_HID = 512
MOL_HID = 256
NUM_TASKS = 12

CHUNK_ATOMS = 4
ROWS = CHUNK_ATOMS * NBR
NPAD = 10560
NCHT = NPAD // CHUNK_ATOMS
NCHPAD = NCHT + 128
B_BASE_CHUNK = 2000
NBUF = 2
A_APW = (96, 416)
B_APW = (32, 128)


def _make_sc_gather_sum(apw0, apw1, call_base_chunk):
    nch0, nch1 = apw0 // CHUNK_ATOMS, apw1 // CHUNK_ATOMS
    nout = 16 * (apw0 + apw1)
    mesh = plsc.VectorSubcoreMesh(core_axis_name="c", subcore_axis_name="s")

    @functools.partial(
        pl.kernel,
        mesh=mesh,
        out_type=jax.ShapeDtypeStruct((nout, H), jnp.float32),
        scratch_types=[
            pltpu.VMEM((max(nch0, nch1), ROWS), jnp.int32),
            pltpu.VMEM_SHARED((16 * max(apw0, apw1), H), jnp.float32),
        ]
        + [pltpu.VMEM((ROWS, H), jnp.float32)] * NBUF
        + [pltpu.VMEM((ROWS,), jnp.int32)] * NBUF
        + [pltpu.SemaphoreType.DMA] * (2 * NBUF),
    )
    def k(idx_hbm, table_hbm, out_hbm, idx_v, acc_sh, *rest):
        rows = rest[:NBUF]
        own = rest[NBUF:2 * NBUF]
        sem_g = rest[2 * NBUF:3 * NBUF]
        sem_s = rest[3 * NBUF:]
        c = lax.axis_index("c")
        s = lax.axis_index("s")
        apw = jnp.where(c == 0, apw0, apw1)
        nch = jnp.where(c == 0, nch0, nch1)
        base_local = s * apw
        base_chunk = call_base_chunk + jnp.where(
            c == 0, s * nch0, 16 * nch0 + s * nch1)
        pltpu.sync_copy(idx_hbm.at[pl.ds(base_chunk, max(nch0, nch1))], idx_v)

        zero = jnp.zeros((16,), jnp.float32)

        def zbuf(r, carry):
            for g in range(H // 16):
                rows[0][r, pl.ds(g * 16, 16)] = zero
            return carry

        lax.fori_loop(0, ROWS, zbuf, 0)

        @pl.when(c == 0)
        def _():
            for t in range(apw0 // ROWS):
                pltpu.sync_copy(
                    rows[0], acc_sh.at[pl.ds(s * apw0 + t * ROWS, ROWS)])
            rem = apw0 % ROWS
            if rem:
                pltpu.sync_copy(
                    rows[0].at[pl.ds(0, rem)],
                    acc_sh.at[pl.ds(s * apw0 + (apw0 // ROWS) * ROWS, rem)])

        @pl.when(c != 0)
        def _():
            for t in range(apw1 // ROWS):
                pltpu.sync_copy(
                    rows[0], acc_sh.at[pl.ds(s * apw1 + t * ROWS, ROWS)])
            rem = apw1 % ROWS
            if rem:
                pltpu.sync_copy(
                    rows[0].at[pl.ds(0, rem)],
                    acc_sh.at[pl.ds(s * apw1 + (apw1 // ROWS) * ROWS, rem)])

        def gather(ci, b):
            pltpu.async_copy(table_hbm.at[idx_v.at[ci]], rows[b], sem_g[b])

        def wait_gather(ci, b):
            pltpu.make_async_copy(table_hbm.at[idx_v.at[ci]], rows[b],
                                  sem_g[b]).wait()

        def scat(b):
            pltpu.async_copy(rows[b], acc_sh.at[own[b]], sem_s[b], add=True)

        def wait_scat(b):
            pltpu.make_async_copy(rows[b], acc_sh.at[own[b]], sem_s[b]).wait()

        for b in range(NBUF):
            gather(b, b)

        def pair(p, carry):
            for b in range(NBUF):
                ci = p * NBUF + b
                wait_gather(ci, b)
                for g in range(ROWS // 16):
                    own[b][pl.ds(g * 16, 16)] = jnp.full(
                        (16,), ci * CHUNK_ATOMS + g // 2, jnp.int32
                    ) + base_local
                scat(b)
            for b in range(NBUF):
                @pl.when(p * NBUF + b + NBUF < nch)
                def _(b=b):
                    ci = p * NBUF + b
                    wait_scat(b)
                    gather(ci + NBUF, b)
            return carry

        lax.fori_loop(0, nch // NBUF, pair, 0)
        for b in range(NBUF):
            wait_scat(b)

        @pl.when(c == 0)
        def _():
            pltpu.sync_copy(acc_sh.at[pl.ds(s * apw0, apw0)],
                            out_hbm.at[pl.ds(s * apw0, apw0)])

        @pl.when(c != 0)
        def _():
            pltpu.sync_copy(acc_sh.at[pl.ds(s * apw1, apw1)],
                            out_hbm.at[pl.ds(16 * apw0 + s * apw1, apw1)])

    return k


ROWS_BLK = 400
NBLK = N_ATOMS // ROWS_BLK
MOLS_BLK = ROWS_BLK // MOL_SIZE


AB_BLK = 20


def _tc_body(orig_ref, aggr_a_ref, aggr_b_ref, w1a_ref, w1b_ref, b1_ref,
             w2_ref, b2_ref, g_ref, b_ref, wm1_ref, bm1_ref, wm2_ref, bm2_ref,
             out_ref, macc):
    i = pl.program_id(0)
    aggr = jnp.where(i < AB_BLK, aggr_a_ref[...], aggr_b_ref[...])
    x = jnp.dot(orig_ref[...], w1a_ref[...], preferred_element_type=jnp.float32)
    x = x + jnp.dot(aggr, w1b_ref[...], preferred_element_type=jnp.float32)
    h = jnp.maximum(x + b1_ref[...], 0.0)
    y = jnp.dot(h, w2_ref[...], preferred_element_type=jnp.float32) + b2_ref[...]
    mu = jnp.mean(y, axis=-1, keepdims=True)
    var = jnp.mean((y - mu) * (y - mu), axis=-1, keepdims=True)
    yn = (y - mu) * lax.rsqrt(var + 1e-5) * g_ref[...] + b_ref[...]
    pooled = jnp.sum(yn.reshape(MOLS_BLK, MOL_SIZE, H), axis=1) * (1.0 / MOL_SIZE)
    macc[pl.ds(i * MOLS_BLK, MOLS_BLK), :] = pooled

    @pl.when(i == NBLK - 1)
    def _():
        m = macc[...]
        hm = jnp.maximum(
            jnp.dot(m, wm1_ref[...], preferred_element_type=jnp.float32)
            + bm1_ref[...], 0.0)
        out = jnp.dot(hm, wm2_ref[...], preferred_element_type=jnp.float32)
        out_ref[...] = (out + bm2_ref[...]) * 0.5


def _tc_ffn(orig, aggr_a, aggr_b, w1a, w1b, b1, w2, b2, g, b, wm1, bm1, wm2,
            bm2):
    full = lambda shape: pl.BlockSpec(shape, lambda i: (0, 0))
    return pl.pallas_call(
        _tc_body,
        grid=(NBLK,),
        in_specs=[
            pl.BlockSpec((ROWS_BLK, H), lambda i: (i, 0)),
            pl.BlockSpec((ROWS_BLK, H), lambda i: (jnp.minimum(i, AB_BLK - 1), 0)),
            pl.BlockSpec((ROWS_BLK, H),
                         lambda i: (jnp.maximum(i - AB_BLK, 0), 0)),
            full((H, FFN_HID)),
            full((H, FFN_HID)),
            full((1, FFN_HID)),
            full((FFN_HID, H)),
            full((1, H)),
            full((1, H)),
            full((1, H)),
            full((H, MOL_HID)),
            full((1, MOL_HID)),
            full((MOL_HID, H)),
            full((1, H)),
        ],
        out_specs=pl.BlockSpec((N_MOLS, H), lambda i: (0, 0)),
        out_shape=jax.ShapeDtypeStruct((N_MOLS, H), jnp.float32),
        scratch_shapes=[pltpu.VMEM((N_MOLS, H), jnp.float32)],
    )(orig, aggr_a, aggr_b, w1a, w1b, b1, w2, b2, g, b, wm1, bm1, wm2, bm2)


def kernel(atom_output, bond_output, original_f_atoms, original_f_bonds,
           a2a, a2b, b2a, b2revb, a_scope,
           W_aa1, b_aa1, W_aa2, b_aa2, ln_aa_g, ln_aa_b,
           W_ab1, b_ab1, W_ab2, b_ab2, ln_ab_g, ln_ab_b,
           W_m1, b_m1, W_m2, b_m2):
    idx = jnp.zeros((NCHPAD, ROWS), jnp.int32).at[:NCHT].set(
        jnp.pad(a2a.astype(jnp.int32),
                ((0, NPAD - N_ATOMS), (0, 0))).reshape(NCHT, ROWS))
    aggr_a = _make_sc_gather_sum(*A_APW, 0)(idx, atom_output)
    aggr_b = _make_sc_gather_sum(*B_APW, B_BASE_CHUNK)(idx, atom_output)

    w1a = W_aa1[:H]
    w1b = W_aa1[H:]
    wm2 = jnp.zeros((MOL_HID, H), jnp.float32).at[:, :NUM_TASKS].set(W_m2)
    bm2 = jnp.zeros((1, H), jnp.float32).at[0, :NUM_TASKS].set(b_m2)
    out = _tc_ffn(original_f_atoms, aggr_a, aggr_b, w1a, w1b,
                  b_aa1.reshape(1, -1), W_aa2, b_aa2.reshape(1, -1),
                  ln_aa_g.reshape(1, -1), ln_aa_b.reshape(1, -1),
                  W_m1, b_m1.reshape(1, -1), wm2, bm2)
    return out[:, :NUM_TASKS]


# speedup vs baseline: 1.7644x; 1.7619x over previous
import functools

import jax
import jax.numpy as jnp
from jax import lax
from jax.experimental import pallas as pl
from jax.experimental.pallas import tpu as pltpu
from jax.experimental.pallas import tpu_sc as plsc

N_ATOMS = 10000
H = 128
NBR = 32
N_MOLS = 200
MOL_SIZE = 50
FFN_HID = 512
MOL_HID = 256
NUM_TASKS = 12

CHUNK_ATOMS = 4
ROWS = CHUNK_ATOMS * NBR
NPAD = 10240
NCHT = NPAD // CHUNK_ATOMS
NCHPAD = NCHT + 128
NBUF = 2
SC_APW = (320, 320)


def _make_sc_gather_sum(apw0, apw1, call_base_chunk):
    nch0, nch1 = apw0 // CHUNK_ATOMS, apw1 // CHUNK_ATOMS
    nout = 16 * (apw0 + apw1)
    mesh = plsc.VectorSubcoreMesh(core_axis_name="c", subcore_axis_name="s")

    @functools.partial(
        pl.kernel,
        mesh=mesh,
        out_type=jax.ShapeDtypeStruct((nout, H), jnp.float32),
        scratch_types=[
            pltpu.VMEM((max(nch0, nch1), ROWS), jnp.int32),
            pltpu.VMEM_SHARED((16 * max(apw0, apw1), H), jnp.float32),
        ]
        + [pltpu.VMEM((ROWS, H), jnp.float32)] * NBUF
        + [pltpu.VMEM((ROWS,), jnp.int32)] * NBUF
        + [pltpu.SemaphoreType.DMA] * (2 * NBUF),
    )
    def k(idx_hbm, table_hbm, out_hbm, idx_v, acc_sh, *rest):
        rows = rest[:NBUF]
        own = rest[NBUF:2 * NBUF]
        sem_g = rest[2 * NBUF:3 * NBUF]
        sem_s = rest[3 * NBUF:]
        c = lax.axis_index("c")
        s = lax.axis_index("s")
        apw = jnp.where(c == 0, apw0, apw1)
        nch = jnp.where(c == 0, nch0, nch1)
        base_local = s * apw
        base_chunk = call_base_chunk + jnp.where(
            c == 0, s * nch0, 16 * nch0 + s * nch1)
        pltpu.sync_copy(idx_hbm.at[pl.ds(base_chunk, max(nch0, nch1))], idx_v)

        zero = jnp.zeros((16,), jnp.float32)

        def zbuf(r, carry):
            for g in range(H // 16):
                rows[0][r, pl.ds(g * 16, 16)] = zero
            return carry

        lax.fori_loop(0, ROWS, zbuf, 0)

        @pl.when(c == 0)
        def _():
            for t in range(apw0 // ROWS):
                pltpu.sync_copy(
                    rows[0], acc_sh.at[pl.ds(s * apw0 + t * ROWS, ROWS)])
            rem = apw0 % ROWS
            if rem:
                pltpu.sync_copy(
                    rows[0].at[pl.ds(0, rem)],
                    acc_sh.at[pl.ds(s * apw0 + (apw0 // ROWS) * ROWS, rem)])

        @pl.when(c != 0)
        def _():
            for t in range(apw1 // ROWS):
                pltpu.sync_copy(
                    rows[0], acc_sh.at[pl.ds(s * apw1 + t * ROWS, ROWS)])
            rem = apw1 % ROWS
            if rem:
                pltpu.sync_copy(
                    rows[0].at[pl.ds(0, rem)],
                    acc_sh.at[pl.ds(s * apw1 + (apw1 // ROWS) * ROWS, rem)])

        def gather(ci, b):
            pltpu.async_copy(table_hbm.at[idx_v.at[ci]], rows[b], sem_g[b])

        def wait_gather(ci, b):
            pltpu.make_async_copy(table_hbm.at[idx_v.at[ci]], rows[b],
                                  sem_g[b]).wait()

        def scat(b):
            pltpu.async_copy(rows[b], acc_sh.at[own[b]], sem_s[b], add=True)

        def wait_scat(b):
            pltpu.make_async_copy(rows[b], acc_sh.at[own[b]], sem_s[b]).wait()

        for b in range(NBUF):
            gather(b, b)

        def pair(p, carry):
            for b in range(NBUF):
                ci = p * NBUF + b
                wait_gather(ci, b)
                for g in range(ROWS // 16):
                    own[b][pl.ds(g * 16, 16)] = jnp.full(
                        (16,), ci * CHUNK_ATOMS + g // 2, jnp.int32
                    ) + base_local
                scat(b)
            for b in range(NBUF):
                @pl.when(p * NBUF + b + NBUF < nch)
                def _(b=b):
                    ci = p * NBUF + b
                    wait_scat(b)
                    gather(ci + NBUF, b)
            return carry

        lax.fori_loop(0, nch // NBUF, pair, 0)
        for b in range(NBUF):
            wait_scat(b)

        @pl.when(c == 0)
        def _():
            pltpu.sync_copy(acc_sh.at[pl.ds(s * apw0, apw0)],
                            out_hbm.at[pl.ds(s * apw0, apw0)])

        @pl.when(c != 0)
        def _():
            pltpu.sync_copy(acc_sh.at[pl.ds(s * apw1, apw1)],
                            out_hbm.at[pl.ds(16 * apw0 + s * apw1, apw1)])

    return k


ROWS_BLK = 400
NBLK = N_ATOMS // ROWS_BLK
MOLS_BLK = ROWS_BLK // MOL_SIZE


def _tc_body(orig_ref, aggr_ref, w1a_ref, w1b_ref, b1_ref,
             w2_ref, b2_ref, g_ref, b_ref, wm1_ref, bm1_ref, wm2_ref, bm2_ref,
             out_ref, macc):
    i = pl.program_id(0)
    x = jnp.dot(orig_ref[...], w1a_ref[...], preferred_element_type=jnp.float32)
    x = x + jnp.dot(aggr_ref[...], w1b_ref[...],
                    preferred_element_type=jnp.float32)
    h = jnp.maximum(x + b1_ref[...], 0.0)
    y = jnp.dot(h, w2_ref[...], preferred_element_type=jnp.float32) + b2_ref[...]
    mu = jnp.mean(y, axis=-1, keepdims=True)
    var = jnp.mean((y - mu) * (y - mu), axis=-1, keepdims=True)
    yn = (y - mu) * lax.rsqrt(var + 1e-5) * g_ref[...] + b_ref[...]
    pooled = jnp.sum(yn.reshape(MOLS_BLK, MOL_SIZE, H), axis=1) * (1.0 / MOL_SIZE)
    macc[pl.ds(i * MOLS_BLK, MOLS_BLK), :] = pooled

    @pl.when(i == NBLK - 1)
    def _():
        m = macc[...]
        hm = jnp.maximum(
            jnp.dot(m, wm1_ref[...], preferred_element_type=jnp.float32)
            + bm1_ref[...], 0.0)
        out = jnp.dot(hm, wm2_ref[...], preferred_element_type=jnp.float32)
        out_ref[...] = (out + bm2_ref[...]) * 0.5


def _tc_ffn(orig, aggr, w1a, w1b, b1, w2, b2, g, b, wm1, bm1, wm2, bm2):
    full = lambda shape: pl.BlockSpec(shape, lambda i: (0, 0))
    return pl.pallas_call(
        _tc_body,
        grid=(NBLK,),
        in_specs=[
            pl.BlockSpec((ROWS_BLK, H), lambda i: (i, 0)),
            pl.BlockSpec((ROWS_BLK, H), lambda i: (i, 0)),
            full((H, FFN_HID)),
            full((H, FFN_HID)),
            full((1, FFN_HID)),
            full((FFN_HID, H)),
            full((1, H)),
            full((1, H)),
            full((1, H)),
            full((H, MOL_HID)),
            full((1, MOL_HID)),
            full((MOL_HID, H)),
            full((1, H)),
        ],
        out_specs=pl.BlockSpec((N_MOLS, H), lambda i: (0, 0)),
        out_shape=jax.ShapeDtypeStruct((N_MOLS, H), jnp.float32),
        scratch_shapes=[pltpu.VMEM((N_MOLS, H), jnp.float32)],
    )(orig, aggr, w1a, w1b, b1, w2, b2, g, b, wm1, bm1, wm2, bm2)


def kernel(atom_output, bond_output, original_f_atoms, original_f_bonds,
           a2a, a2b, b2a, b2revb, a_scope,
           W_aa1, b_aa1, W_aa2, b_aa2, ln_aa_g, ln_aa_b,
           W_ab1, b_ab1, W_ab2, b_ab2, ln_ab_g, ln_ab_b,
           W_m1, b_m1, W_m2, b_m2):
    idx = jnp.concatenate(
        [a2a.astype(jnp.int32).reshape(N_ATOMS * NBR // ROWS, ROWS),
         jnp.zeros((NCHPAD - N_ATOMS * NBR // ROWS, ROWS), jnp.int32)])
    aggr = _make_sc_gather_sum(*SC_APW, 0)(idx, atom_output)

    w1a = W_aa1[:H]
    w1b = W_aa1[H:]
    wm2 = jnp.zeros((MOL_HID, H), jnp.float32).at[:, :NUM_TASKS].set(W_m2)
    bm2 = jnp.zeros((1, H), jnp.float32).at[0, :NUM_TASKS].set(b_m2)
    out = _tc_ffn(original_f_atoms, aggr, w1a, w1b,
                  b_aa1.reshape(1, -1), W_aa2, b_aa2.reshape(1, -1),
                  ln_aa_g.reshape(1, -1), ln_aa_b.reshape(1, -1),
                  W_m1, b_m1.reshape(1, -1), wm2, bm2)
    return out[:, :NUM_TASKS]


# speedup vs baseline: 2.7307x; 1.5477x over previous
import functools

import jax
import jax.numpy as jnp
from jax import lax
from jax.experimental import pallas as pl
from jax.experimental.pallas import tpu as pltpu
from jax.experimental.pallas import tpu_sc as plsc

N_ATOMS = 10000
H = 128
NBR = 32
N_MOLS = 200
MOL_SIZE = 50
FFN_HID = 512
MOL_HID = 256
NUM_TASKS = 12

CHUNK_ATOMS = 4
ROWS = CHUNK_ATOMS * NBR
NCHT = N_ATOMS * NBR // ROWS
NCHPAD = 3200
NBUF = 2
SC_APW = (96, 96)
CALL_BASES = (0, 2400, 4800, 7200, 9600)


def _make_sc_gather_sum(apw0, apw1, call_base_chunk):
    nch0, nch1 = apw0 // CHUNK_ATOMS, apw1 // CHUNK_ATOMS
    nout = 16 * (apw0 + apw1)
    mesh = plsc.VectorSubcoreMesh(core_axis_name="c", subcore_axis_name="s")

    @functools.partial(
        pl.kernel,
        mesh=mesh,
        out_type=jax.ShapeDtypeStruct((nout, H), jnp.float32),
        scratch_types=[
            pltpu.VMEM((max(nch0, nch1), ROWS), jnp.int32),
            pltpu.VMEM_SHARED((16 * max(apw0, apw1), H), jnp.float32),
            pltpu.VMEM_SHARED((N_ATOMS, H), jnp.float32),
        ]
        + [pltpu.VMEM((ROWS, H), jnp.float32)] * NBUF
        + [pltpu.VMEM((ROWS,), jnp.int32)] * NBUF
        + [pltpu.SemaphoreType.DMA] * (2 * NBUF),
    )
    def k(idx_hbm, table_hbm, out_hbm, idx_v, acc_sh, table_sh, *rest):
        rows = rest[:NBUF]
        own = rest[NBUF:2 * NBUF]
        sem_g = rest[2 * NBUF:3 * NBUF]
        sem_s = rest[3 * NBUF:]
        c = lax.axis_index("c")
        s = lax.axis_index("s")
        apw = jnp.where(c == 0, apw0, apw1)
        nch = jnp.where(c == 0, nch0, nch1)
        base_local = s * apw
        base_chunk = call_base_chunk + jnp.where(
            c == 0, s * nch0, 16 * nch0 + s * nch1)
        pltpu.sync_copy(idx_hbm.at[pl.ds(base_chunk, max(nch0, nch1))], idx_v)

        zero = jnp.zeros((16,), jnp.float32)

        def zbuf(r, carry):
            for g in range(H // 16):
                rows[0][r, pl.ds(g * 16, 16)] = zero
            return carry

        lax.fori_loop(0, ROWS, zbuf, 0)

        @pl.when(c == 0)
        def _():
            for t in range(apw0 // ROWS):
                pltpu.sync_copy(
                    rows[0], acc_sh.at[pl.ds(s * apw0 + t * ROWS, ROWS)])
            rem = apw0 % ROWS
            if rem:
                pltpu.sync_copy(
                    rows[0].at[pl.ds(0, rem)],
                    acc_sh.at[pl.ds(s * apw0 + (apw0 // ROWS) * ROWS, rem)])

        @pl.when(c != 0)
        def _():
            for t in range(apw1 // ROWS):
                pltpu.sync_copy(
                    rows[0], acc_sh.at[pl.ds(s * apw1 + t * ROWS, ROWS)])
            rem = apw1 % ROWS
            if rem:
                pltpu.sync_copy(
                    rows[0].at[pl.ds(0, rem)],
                    acc_sh.at[pl.ds(s * apw1 + (apw1 // ROWS) * ROWS, rem)])

        tpw = 624
        pltpu.sync_copy(table_hbm.at[pl.ds(s * tpw, tpw)],
                        table_sh.at[pl.ds(s * tpw, tpw)])

        @pl.when(s == 0)
        def _():
            pltpu.sync_copy(table_hbm.at[pl.ds(16 * tpw, N_ATOMS - 16 * tpw)],
                            table_sh.at[pl.ds(16 * tpw, N_ATOMS - 16 * tpw)])

        plsc.subcore_barrier()

        def gather(ci, b):
            pltpu.async_copy(table_sh.at[idx_v.at[ci]], rows[b], sem_g[b])

        def wait_gather(ci, b):
            pltpu.make_async_copy(table_sh.at[idx_v.at[ci]], rows[b],
                                  sem_g[b]).wait()

        def scat(b):
            pltpu.async_copy(rows[b], acc_sh.at[own[b]], sem_s[b], add=True)

        def wait_scat(b):
            pltpu.make_async_copy(rows[b], acc_sh.at[own[b]], sem_s[b]).wait()

        for b in range(NBUF):
            gather(b, b)

        def pair(p, carry):
            for b in range(NBUF):
                ci = p * NBUF + b
                wait_gather(ci, b)
                for g in range(ROWS // 16):
                    own[b][pl.ds(g * 16, 16)] = jnp.full(
                        (16,), ci * CHUNK_ATOMS + g // 2, jnp.int32
                    ) + base_local
                scat(b)
            for b in range(NBUF):
                @pl.when(p * NBUF + b + NBUF < nch)
                def _(b=b):
                    ci = p * NBUF + b
                    wait_scat(b)
                    gather(ci + NBUF, b)
            return carry

        lax.fori_loop(0, nch // NBUF, pair, 0)
        for b in range(NBUF):
            wait_scat(b)

        @pl.when(c == 0)
        def _():
            pltpu.sync_copy(acc_sh.at[pl.ds(s * apw0, apw0)],
                            out_hbm.at[pl.ds(s * apw0, apw0)])

        @pl.when(c != 0)
        def _():
            pltpu.sync_copy(acc_sh.at[pl.ds(s * apw1, apw1)],
                            out_hbm.at[pl.ds(16 * apw0 + s * apw1, apw1)])

    return k


ROWS_BLK = 400
NBLK = N_ATOMS // ROWS_BLK
MOLS_BLK = ROWS_BLK // MOL_SIZE


def _tc_body(orig_ref, aggr_a_ref, aggr_b_ref, aggr_c_ref, aggr_d_ref,
             aggr_e_ref, w1a_ref, w1b_ref,
             b1_ref, w2_ref, b2_ref, g_ref, b_ref, wm1_ref, bm1_ref, wm2_ref,
             bm2_ref, out_ref, macc):
    i = pl.program_id(0)
    aggr = jnp.where(
        i < 6, aggr_a_ref[...],
        jnp.where(i < 12, aggr_b_ref[...],
                  jnp.where(i < 18, aggr_c_ref[...],
                            jnp.where(i < 24, aggr_d_ref[...],
                                      aggr_e_ref[...]))))
    x = jnp.dot(orig_ref[...], w1a_ref[...], preferred_element_type=jnp.float32)
    x = x + jnp.dot(aggr, w1b_ref[...],
                    preferred_element_type=jnp.float32)
    h = jnp.maximum(x + b1_ref[...], 0.0)
    y = jnp.dot(h, w2_ref[...], preferred_element_type=jnp.float32) + b2_ref[...]
    mu = jnp.mean(y, axis=-1, keepdims=True)
    var = jnp.mean((y - mu) * (y - mu), axis=-1, keepdims=True)
    yn = (y - mu) * lax.rsqrt(var + 1e-5) * g_ref[...] + b_ref[...]
    pooled = jnp.sum(yn.reshape(MOLS_BLK, MOL_SIZE, H), axis=1) * (1.0 / MOL_SIZE)
    macc[pl.ds(i * MOLS_BLK, MOLS_BLK), :] = pooled

    @pl.when(i == NBLK - 1)
    def _():
        m = macc[...]
        hm = jnp.maximum(
            jnp.dot(m, wm1_ref[...], preferred_element_type=jnp.float32)
            + bm1_ref[...], 0.0)
        out = jnp.dot(hm, wm2_ref[...], preferred_element_type=jnp.float32)
        out_ref[...] = (out + bm2_ref[...]) * 0.5


def _tc_ffn(orig, aggrs, w1a, w1b, b1, w2, b2, g, b, wm1, bm1, wm2, bm2):
    full = lambda shape: pl.BlockSpec(shape, lambda i: (0, 0))
    return pl.pallas_call(
        _tc_body,
        grid=(NBLK,),
        in_specs=[
            pl.BlockSpec((ROWS_BLK, H), lambda i: (i, 0)),
        ] + [
            pl.BlockSpec((ROWS_BLK, H),
                         functools.partial(
                             lambda k, i: (jnp.clip(i - 6 * k, 0, 5), 0), k))
            for k in range(5)
        ] + [
            full((H, FFN_HID)),
            full((H, FFN_HID)),
            full((1, FFN_HID)),
            full((FFN_HID, H)),
            full((1, H)),
            full((1, H)),
            full((1, H)),
            full((H, MOL_HID)),
            full((1, MOL_HID)),
            full((MOL_HID, H)),
            full((1, H)),
        ],
        out_specs=pl.BlockSpec((N_MOLS, H), lambda i: (0, 0)),
        out_shape=jax.ShapeDtypeStruct((N_MOLS, H), jnp.float32),
        scratch_shapes=[pltpu.VMEM((N_MOLS, H), jnp.float32)],
    )(orig, *aggrs, w1a, w1b, b1, w2, b2, g, b, wm1, bm1, wm2, bm2)


def kernel(atom_output, bond_output, original_f_atoms, original_f_bonds,
           a2a, a2b, b2a, b2revb, a_scope,
           W_aa1, b_aa1, W_aa2, b_aa2, ln_aa_g, ln_aa_b,
           W_ab1, b_ab1, W_ab2, b_ab2, ln_ab_g, ln_ab_b,
           W_m1, b_m1, W_m2, b_m2):
    idx = jnp.concatenate(
        [a2a.astype(jnp.int32).reshape(NCHT, ROWS),
         jnp.zeros((NCHPAD - NCHT, ROWS), jnp.int32)])
    aggrs = [_make_sc_gather_sum(*SC_APW, base // 4)(idx, atom_output)
             for base in CALL_BASES]

    w1a = W_aa1[:H]
    w1b = W_aa1[H:]
    wm2 = jnp.zeros((MOL_HID, H), jnp.float32).at[:, :NUM_TASKS].set(W_m2)
    bm2 = jnp.zeros((1, H), jnp.float32).at[0, :NUM_TASKS].set(b_m2)
    out = _tc_ffn(original_f_atoms, aggrs, w1a, w1b,
                  b_aa1.reshape(1, -1), W_aa2, b_aa2.reshape(1, -1),
                  ln_aa_g.reshape(1, -1), ln_aa_b.reshape(1, -1),
                  W_m1, b_m1.reshape(1, -1), wm2, bm2)
    return out[:, :NUM_TASKS]


# speedup vs baseline: 3.3632x; 1.2316x over previous
import functools

import jax
import jax.numpy as jnp
from jax import lax
from jax.experimental import pallas as pl
from jax.experimental.pallas import tpu as pltpu
from jax.experimental.pallas import tpu_sc as plsc

N_ATOMS = 10000
H = 128
NBR = 32
N_MOLS = 200
MOL_SIZE = 50
FFN_HID = 512
MOL_HID = 256
NUM_TASKS = 12

CHUNK_ATOMS = 4
ROWS = CHUNK_ATOMS * NBR
NCHT = N_ATOMS * NBR // ROWS
NCHPAD = 3200
NBUF = 2
SC_APW = (96, 96)
CALL_BASES = (0, 2400, 4800, 7200)


def _make_sc_gather_sum(apw0, apw1, call_base_chunk):
    nch0, nch1 = apw0 // CHUNK_ATOMS, apw1 // CHUNK_ATOMS
    nout = 16 * (apw0 + apw1)
    mesh = plsc.VectorSubcoreMesh(core_axis_name="c", subcore_axis_name="s")

    @functools.partial(
        pl.kernel,
        mesh=mesh,
        out_type=jax.ShapeDtypeStruct((nout, H), jnp.float32),
        scratch_types=[
            pltpu.VMEM((max(nch0, nch1), ROWS), jnp.int32),
            pltpu.VMEM_SHARED((16 * max(apw0, apw1), H), jnp.float32),
            pltpu.VMEM_SHARED((N_ATOMS, H), jnp.float32),
        ]
        + [pltpu.VMEM((ROWS, H), jnp.float32)] * NBUF
        + [pltpu.VMEM((ROWS,), jnp.int32)] * NBUF
        + [pltpu.SemaphoreType.DMA] * (2 * NBUF),
    )
    def k(idx_hbm, table_hbm, out_hbm, idx_v, acc_sh, table_sh, *rest):
        rows = rest[:NBUF]
        own = rest[NBUF:2 * NBUF]
        sem_g = rest[2 * NBUF:3 * NBUF]
        sem_s = rest[3 * NBUF:]
        c = lax.axis_index("c")
        s = lax.axis_index("s")
        apw = jnp.where(c == 0, apw0, apw1)
        nch = jnp.where(c == 0, nch0, nch1)
        base_local = s * apw
        base_chunk = call_base_chunk + jnp.where(
            c == 0, s * nch0, 16 * nch0 + s * nch1)
        pltpu.sync_copy(idx_hbm.at[pl.ds(base_chunk, max(nch0, nch1))], idx_v)

        zero = jnp.zeros((16,), jnp.float32)

        def zbuf(r, carry):
            for g in range(H // 16):
                rows[0][r, pl.ds(g * 16, 16)] = zero
            return carry

        lax.fori_loop(0, ROWS, zbuf, 0)

        @pl.when(c == 0)
        def _():
            for t in range(apw0 // ROWS):
                pltpu.sync_copy(
                    rows[0], acc_sh.at[pl.ds(s * apw0 + t * ROWS, ROWS)])
            rem = apw0 % ROWS
            if rem:
                pltpu.sync_copy(
                    rows[0].at[pl.ds(0, rem)],
                    acc_sh.at[pl.ds(s * apw0 + (apw0 // ROWS) * ROWS, rem)])

        @pl.when(c != 0)
        def _():
            for t in range(apw1 // ROWS):
                pltpu.sync_copy(
                    rows[0], acc_sh.at[pl.ds(s * apw1 + t * ROWS, ROWS)])
            rem = apw1 % ROWS
            if rem:
                pltpu.sync_copy(
                    rows[0].at[pl.ds(0, rem)],
                    acc_sh.at[pl.ds(s * apw1 + (apw1 // ROWS) * ROWS, rem)])

        tpw = 624
        pltpu.sync_copy(table_hbm.at[pl.ds(s * tpw, tpw)],
                        table_sh.at[pl.ds(s * tpw, tpw)])

        @pl.when(s == 0)
        def _():
            pltpu.sync_copy(table_hbm.at[pl.ds(16 * tpw, N_ATOMS - 16 * tpw)],
                            table_sh.at[pl.ds(16 * tpw, N_ATOMS - 16 * tpw)])

        plsc.subcore_barrier()

        def gather(ci, b):
            pltpu.async_copy(table_sh.at[idx_v.at[ci]], rows[b], sem_g[b])

        def wait_gather(ci, b):
            pltpu.make_async_copy(table_sh.at[idx_v.at[ci]], rows[b],
                                  sem_g[b]).wait()

        def scat(b):
            pltpu.async_copy(rows[b], acc_sh.at[own[b]], sem_s[b], add=True)

        def wait_scat(b):
            pltpu.make_async_copy(rows[b], acc_sh.at[own[b]], sem_s[b]).wait()

        for b in range(NBUF):
            gather(b, b)

        def pair(p, carry):
            for b in range(NBUF):
                ci = p * NBUF + b
                wait_gather(ci, b)
                for g in range(ROWS // 16):
                    own[b][pl.ds(g * 16, 16)] = jnp.full(
                        (16,), ci * CHUNK_ATOMS + g // 2, jnp.int32
                    ) + base_local
                scat(b)
            for b in range(NBUF):
                @pl.when(p * NBUF + b + NBUF < nch)
                def _(b=b):
                    ci = p * NBUF + b
                    wait_scat(b)
                    gather(ci + NBUF, b)
            return carry

        lax.fori_loop(0, nch // NBUF, pair, 0)
        for b in range(NBUF):
            wait_scat(b)

        @pl.when(c == 0)
        def _():
            pltpu.sync_copy(acc_sh.at[pl.ds(s * apw0, apw0)],
                            out_hbm.at[pl.ds(s * apw0, apw0)])

        @pl.when(c != 0)
        def _():
            pltpu.sync_copy(acc_sh.at[pl.ds(s * apw1, apw1)],
                            out_hbm.at[pl.ds(16 * apw0 + s * apw1, apw1)])

    return k


ROWS_BLK = 400
NBLK = N_ATOMS // ROWS_BLK
MOLS_BLK = ROWS_BLK // MOL_SIZE


def _tc_body(orig_ref, aggr_a_ref, aggr_b_ref, aggr_c_ref, aggr_d_ref,
             w1a_ref, w1b_ref,
             b1_ref, w2_ref, b2_ref, g_ref, b_ref, wm1_ref, bm1_ref, wm2_ref,
             bm2_ref, out_ref, macc):
    i = pl.program_id(0)
    aggr = jnp.where(
        i < 6, aggr_a_ref[...],
        jnp.where(i < 12, aggr_b_ref[...],
                  jnp.where(i < 18, aggr_c_ref[...], aggr_d_ref[...])))
    x = jnp.dot(orig_ref[...], w1a_ref[...], preferred_element_type=jnp.float32)
    x = x + jnp.dot(aggr, w1b_ref[...],
                    preferred_element_type=jnp.float32)
    h = jnp.maximum(x + b1_ref[...], 0.0)
    y = jnp.dot(h, w2_ref[...], preferred_element_type=jnp.float32) + b2_ref[...]
    mu = jnp.mean(y, axis=-1, keepdims=True)
    var = jnp.mean((y - mu) * (y - mu), axis=-1, keepdims=True)
    yn = (y - mu) * lax.rsqrt(var + 1e-5) * g_ref[...] + b_ref[...]
    pooled = jnp.sum(yn.reshape(MOLS_BLK, MOL_SIZE, H), axis=1) * (1.0 / MOL_SIZE)
    macc[pl.ds(i * MOLS_BLK, MOLS_BLK), :] = pooled

    @pl.when(i == NBLK - 1)
    def _():
        m = macc[...]
        hm = jnp.maximum(
            jnp.dot(m, wm1_ref[...], preferred_element_type=jnp.float32)
            + bm1_ref[...], 0.0)
        out = jnp.dot(hm, wm2_ref[...], preferred_element_type=jnp.float32)
        out_ref[...] = (out + bm2_ref[...]) * 0.5


def _tc_ffn(orig, aggrs, w1a, w1b, b1, w2, b2, g, b, wm1, bm1, wm2, bm2):
    full = lambda shape: pl.BlockSpec(shape, lambda i: (0, 0))
    return pl.pallas_call(
        _tc_body,
        grid=(NBLK,),
        in_specs=[
            pl.BlockSpec((ROWS_BLK, H), lambda i: (i, 0)),
        ] + [
            pl.BlockSpec((ROWS_BLK, H),
                         functools.partial(
                             lambda k, i: (jnp.clip(i - 6 * k, 0,
                                                    6 if k == 3 else 5), 0),
                             k))
            for k in range(4)
        ] + [
            full((H, FFN_HID)),
            full((H, FFN_HID)),
            full((1, FFN_HID)),
            full((FFN_HID, H)),
            full((1, H)),
            full((1, H)),
            full((1, H)),
            full((H, MOL_HID)),
            full((1, MOL_HID)),
            full((MOL_HID, H)),
            full((1, H)),
        ],
        out_specs=pl.BlockSpec((N_MOLS, H), lambda i: (0, 0)),
        out_shape=jax.ShapeDtypeStruct((N_MOLS, H), jnp.float32),
        scratch_shapes=[pltpu.VMEM((N_MOLS, H), jnp.float32)],
    )(orig, *aggrs, w1a, w1b, b1, w2, b2, g, b, wm1, bm1, wm2, bm2)


def kernel(atom_output, bond_output, original_f_atoms, original_f_bonds,
           a2a, a2b, b2a, b2revb, a_scope,
           W_aa1, b_aa1, W_aa2, b_aa2, ln_aa_g, ln_aa_b,
           W_ab1, b_ab1, W_ab2, b_ab2, ln_ab_g, ln_ab_b,
           W_m1, b_m1, W_m2, b_m2):
    idx = jnp.concatenate(
        [a2a.astype(jnp.int32).reshape(NCHT, ROWS),
         jnp.zeros((NCHPAD - NCHT, ROWS), jnp.int32)])
    aggrs = [_make_sc_gather_sum(*SC_APW, base // 4)(idx, atom_output)
             for base in CALL_BASES]

    w1a = W_aa1[:H]
    w1b = W_aa1[H:]
    wm2 = jnp.zeros((MOL_HID, H), jnp.float32).at[:, :NUM_TASKS].set(W_m2)
    bm2 = jnp.zeros((1, H), jnp.float32).at[0, :NUM_TASKS].set(b_m2)
    out = _tc_ffn(original_f_atoms, aggrs, w1a, w1b,
                  b_aa1.reshape(1, -1), W_aa2, b_aa2.reshape(1, -1),
                  ln_aa_g.reshape(1, -1), ln_aa_b.reshape(1, -1),
                  W_m1, b_m1.reshape(1, -1), wm2, bm2)
    return out[:, :NUM_TASKS]
